# baseline (speedup 1.0000x reference)
import jax
import jax.numpy as jnp
from jax import lax
from jax.experimental import pallas as pl
from jax.experimental.pallas import tpu as pltpu
from jax.experimental.pallas import tpu_sc as plsc

N = 10000
E = 160000
D = 256
NT = 16
NB = 162
B = 64
W = 128
NC1 = 4
NC2 = 2
EPT = NB * B
EPAD = NT * EPT
NP = 10240
RPT = NP // NT

_mesh = plsc.VectorSubcoreMesh(core_axis_name="c", subcore_axis_name="s")


def _zero_rows(ref, rows):
    z = jnp.zeros((16,), jnp.float32)

    def body(i, _):
        for k in range(W // 16):
            ref[i, pl.ds(k * 16, 16)] = z
        return 0

    lax.fori_loop(0, rows, body, 0)


def _unpack_dst(pk_v, j, dstb_v, base):
    for k in range(B // 16):
        p16 = pk_v[pl.ds(j * B + k * 16, 16)]
        dstb_v[pl.ds(k * 16, 16)] = (p16 & 0xFFFF) + base


def _unpack_src(pk_v, j, srcb_v):
    for k in range(B // 16):
        p16 = pk_v[pl.ds(j * B + k * 16, 16)]
        srcb_v[pl.ds(k * 16, 16)] = p16 >> 16


def _scale(w_v, j, gbuf, sbuf):
    def body(g, _):
        wv = w_v[pl.ds(j * B + g * 8, 16)]
        for l in range(8):
            w = wv[l]
            e = g * 8 + l
            for k in range(W // 16):
                sl = pl.ds(k * 16, 16)
                sbuf[e, sl] = gbuf[e, sl] * w
        return 0

    lax.fori_loop(0, B // 8, body, 0)


def _spmm_round(pk_v, w_v, dst3, src3, gb, gsem, ssem,
                h_hbm, acc_sh, base):
    def g_start(t):
        pltpu.async_copy(h_hbm.at[dst3[t % 3]], gb[t % 3], gsem[t % 3])

    def g_wait(t):
        pltpu.make_async_copy(h_hbm.at[dst3[t % 3]], gb[t % 3],
                              gsem[t % 3]).wait()

    def s_start(t):
        pltpu.async_copy(gb[t % 3], acc_sh.at[src3[t % 3]], ssem[t % 3],
                         add=True)

    def s_wait(t):
        pltpu.make_async_copy(gb[t % 3], acc_sh.at[src3[t % 3]],
                              ssem[t % 3]).wait()

    _unpack_dst(pk_v, 0, dst3[0], base)
    g_start(0)

    def triple(q, _):
        j0 = 3 * q
        for t in range(3):
            j = j0 + t

            @pl.when(j >= 2)
            def _():
                s_wait(t + 1)

            @pl.when(j + 1 < NB)
            def _():
                _unpack_dst(pk_v, j + 1, dst3[(t + 1) % 3], base)
                g_start(t + 1)

            g_wait(t)
            _scale(w_v, j, gb[t % 3], gb[t % 3])
            _unpack_src(pk_v, j, src3[t % 3])
            s_start(t)
        return 0

    lax.fori_loop(0, NB // 3, triple, 0)
    s_wait(1)
    s_wait(2)


def _zero_acc_slice(gbuf, acc_sh, s):
    _zero_rows(gbuf, B)
    for i in range(RPT // B):
        pltpu.sync_copy(gbuf, acc_sh.at[pl.ds(s * RPT + i * B, B)])
    rem = RPT - (RPT // B) * B
    if rem:
        pltpu.sync_copy(gbuf.at[pl.ds(0, rem)],
                        acc_sh.at[pl.ds(s * RPT + (RPT // B) * B, rem)])


def _sc_degw(pk_hbm, ew_hbm, w_hbm,
             deg_sh, pk_v, w_v, dst0, dtmp_v, ones_v, zb):
    c = lax.axis_index("c")
    s = lax.axis_index("s")

    pltpu.sync_copy(pk_hbm.at[s], pk_v)
    pltpu.sync_copy(ew_hbm.at[s], w_v.at[pl.ds(0, EPT)])

    _zero_rows(zb, B)
    for i in range(RPT // W):
        pltpu.sync_copy(zb.at[i], deg_sh.at[pl.ds(s * RPT + i * W, W)])

    def fill_ones(i, _):
        ones_v[pl.ds(i * 16, 16)] = jnp.full((16,), 1.0, jnp.float32)
        return 0

    lax.fori_loop(0, B // 16, fill_ones, 0)
    plsc.subcore_barrier()

    def deg_body(j, _):
        _unpack_dst(pk_v, j, dst0, 0)
        pltpu.sync_copy(ones_v, deg_sh.at[dst0], add=True)
        return 0

    lax.fori_loop(0, NB, deg_body, 0)
    plsc.subcore_barrier()

    def w_body(j, _):
        _unpack_dst(pk_v, j, dst0, 0)
        pltpu.sync_copy(deg_sh.at[dst0], dtmp_v)
        for k in range(B // 16):
            sl = pl.ds(j * B + k * 16, 16)
            d16 = jnp.maximum(dtmp_v[pl.ds(k * 16, 16)], 1.0)
            w_v[sl] = w_v[sl] / d16
        return 0

    lax.fori_loop(0, NB, w_body, 0)

    @pl.when(c == 0)
    def _():
        pltpu.sync_copy(w_v.at[pl.ds(0, EPT)], w_hbm.at[s])


def _sc_all(pk_hbm, w_hbm, h_hbm, f1_hbm, f2_hbm,
            acc_sh, pk_v, w_v, dst0, dst1, dst2,
            src0, src1, src2,
            gb0, gb1, gb2, gsem0, gsem1, gsem2, ssem0, ssem1, ssem2):
    c = lax.axis_index("c")
    s = lax.axis_index("s")
    dst3 = (dst0, dst1, dst2)
    src3 = (src0, src1, src2)
    gb = (gb0, gb1, gb2)
    gsem = (gsem0, gsem1, gsem2)
    ssem = (ssem0, ssem1, ssem2)

    pltpu.sync_copy(pk_hbm.at[s], pk_v)
    pltpu.sync_copy(w_hbm.at[s], w_v.at[pl.ds(0, EPT)])

    for r in range(2):
        base = (c + 2 * r) * NP
        _zero_acc_slice(gb0, acc_sh, s)
        plsc.subcore_barrier()
        _spmm_round(pk_v, w_v, dst3, src3, gb, gsem, ssem,
                    h_hbm, acc_sh, base)
        plsc.subcore_barrier()
        pltpu.sync_copy(acc_sh.at[pl.ds(s * RPT, RPT)],
                        f1_hbm.at[pl.ds(base + s * RPT, RPT)])
    plsc.subcore_barrier()

    _zero_acc_slice(gb0, acc_sh, s)
    plsc.subcore_barrier()
    _spmm_round(pk_v, w_v, dst3, src3, gb, gsem, ssem,
                f1_hbm, acc_sh, (2 + c) * NP)
    plsc.subcore_barrier()
    pltpu.sync_copy(acc_sh.at[pl.ds(s * RPT, RPT)],
                    f2_hbm.at[pl.ds(c * NP + s * RPT, RPT)])


_sc_scratch_common = [
    pltpu.VMEM((EPT,), jnp.int32),
    pltpu.VMEM((EPT + 128,), jnp.float32),
    pltpu.VMEM((B,), jnp.int32),
    pltpu.VMEM((B,), jnp.int32),
    pltpu.VMEM((B,), jnp.int32),
    pltpu.VMEM((B,), jnp.int32),
    pltpu.VMEM((B,), jnp.int32),
    pltpu.VMEM((B,), jnp.int32),
]

_sc_bufs = [
    pltpu.VMEM((B, W), jnp.float32),
    pltpu.VMEM((B, W), jnp.float32),
    pltpu.VMEM((B, W), jnp.float32),
    pltpu.SemaphoreType.DMA,
    pltpu.SemaphoreType.DMA,
    pltpu.SemaphoreType.DMA,
    pltpu.SemaphoreType.DMA,
    pltpu.SemaphoreType.DMA,
    pltpu.SemaphoreType.DMA,
]

_sc_degw_call = pl.kernel(
    _sc_degw,
    out_type=jax.ShapeDtypeStruct((NT, EPT), jnp.float32),
    mesh=_mesh,
    scratch_types=[
        pltpu.VMEM_SHARED((NP,), jnp.float32),
        pltpu.VMEM((EPT,), jnp.int32),
        pltpu.VMEM((EPT + 128,), jnp.float32),
        pltpu.VMEM((B,), jnp.int32),
        pltpu.VMEM((B,), jnp.float32),
        pltpu.VMEM((B,), jnp.float32),
        pltpu.VMEM((B, W), jnp.float32),
    ],
)

_sc_all_call = pl.kernel(
    _sc_all,
    out_type=[
        jax.ShapeDtypeStruct((NC1 * NP, W), jnp.float32),
        jax.ShapeDtypeStruct((NC2 * NP, W), jnp.float32),
    ],
    mesh=_mesh,
    scratch_types=(
        [pltpu.VMEM_SHARED((NP, W), jnp.float32)]
        + _sc_scratch_common
        + _sc_bufs
    ),
)


def _mm_body(x_ref, wt_ref, o_ref):
    o_ref[pl.ds(0, N), :] = jax.lax.dot(
        x_ref[...], wt_ref[...],
        precision=jax.lax.Precision.HIGHEST,
        preferred_element_type=jnp.float32)
    o_ref[pl.ds(N, NP - N), :] = jnp.zeros((NP - N, W), jnp.float32)


def _comb_body(m_ref, f1_ref, f2_ref, h_ref, o_ref):
    scale = 1.0 / 3.0
    m0 = m_ref[0] * scale
    m1 = m_ref[1] * scale
    m2 = m_ref[2] * scale
    o_ref[...] = (m0 * f1_ref[pl.ds(0, N), :]
                  + m1 * f2_ref[pl.ds(0, N), :]
                  + m2 * h_ref[pl.ds(0, N), :])


def kernel(x, edge_index, edge_weight, W0, W1, merger):
    src = edge_index[0]
    dst = edge_index[1]
    pad = EPAD - E
    ar = jnp.arange(pad, dtype=jnp.int32)
    padv = N + (ar % (NP - N))
    dst_p = jnp.concatenate([dst, padv])
    src_p = jnp.concatenate([src, padv])
    packed = (dst_p | (src_p << 16)).reshape(NT, EPT)
    ew_p = jnp.concatenate(
        [edge_weight, jnp.zeros((pad,), jnp.float32)]).reshape(NT, EPT)

    wt = jnp.concatenate([W0.T, W1.T], axis=1)

    h_flat = pl.pallas_call(
        _mm_body,
        grid=(NC1,),
        in_specs=[
            pl.BlockSpec((N, D), lambda c: (0, 0)),
            pl.BlockSpec((D, W), lambda c: (0, c)),
        ],
        out_specs=pl.BlockSpec((NP, W), lambda c: (c, 0)),
        out_shape=jax.ShapeDtypeStruct((NC1 * NP, W), jnp.float32),
    )(x, wt)

    w_pad = _sc_degw_call(packed, ew_p)
    f1_flat, f2_flat = _sc_all_call(packed, w_pad, h_flat)

    out = pl.pallas_call(
        _comb_body,
        grid=(NC2,),
        in_specs=[
            pl.BlockSpec(memory_space=pltpu.SMEM),
            pl.BlockSpec((NP, W), lambda k: (k, 0)),
            pl.BlockSpec((NP, W), lambda k: (k, 0)),
            pl.BlockSpec((NP, W), lambda k: (2 + k, 0)),
        ],
        out_specs=pl.BlockSpec((N, W), lambda k: (0, k)),
        out_shape=jax.ShapeDtypeStruct((N, D), jnp.float32),
    )(merger, f1_flat, f2_flat, h_flat)
    return out


# speedup vs baseline: 1.5679x; 1.5679x over previous
import jax
import jax.numpy as jnp
from jax import lax
from jax.experimental import pallas as pl
from jax.experimental.pallas import tpu as pltpu
from jax.experimental.pallas import tpu_sc as plsc

N = 10000
E = 160000
D = 256
NT = 16
NB = 216
B = 48
W = 128
NC1 = 4
NC2 = 2
EPT = NB * B
EPAD = NT * EPT
NP = 10240
RPT = NP // NT

_mesh = plsc.VectorSubcoreMesh(core_axis_name="c", subcore_axis_name="s")


def _zero_rows(ref, rows):
    z = jnp.zeros((16,), jnp.float32)

    def body(i, _):
        for k in range(W // 16):
            ref[i, pl.ds(k * 16, 16)] = z
        return 0

    lax.fori_loop(0, rows, body, 0)


def _unpack_dst(pk_v, j, dstb_v, base):
    for k in range(B // 16):
        p16 = pk_v[pl.ds(j * B + k * 16, 16)]
        dstb_v[pl.ds(k * 16, 16)] = (p16 & 0xFFFF) + base


def _unpack_src(pk_v, j, srcb_v):
    for k in range(B // 16):
        p16 = pk_v[pl.ds(j * B + k * 16, 16)]
        srcb_v[pl.ds(k * 16, 16)] = p16 >> 16


def _scale(w_v, j, gbuf, sbuf):
    def body(g, _):
        wv = w_v[pl.ds(j * B + g * 8, 16)]
        for l in range(8):
            w = wv[l]
            e = g * 8 + l
            for k in range(W // 16):
                sl = pl.ds(k * 16, 16)
                sbuf[e, sl] = gbuf[e, sl] * w
        return 0

    lax.fori_loop(0, B // 8, body, 0)


def _spmm_round(pk_v, w_v, dst2, src2, gb, sb, gsem, ssem,
                h_hbm, acc_sh, base):
    def g_start(t):
        pltpu.async_copy(h_hbm.at[dst2[t % 2]], gb[t % 2], gsem[t % 2])

    def g_wait(t):
        pltpu.make_async_copy(h_hbm.at[dst2[t % 2]], gb[t % 2],
                              gsem[t % 2]).wait()

    def s_start(t):
        pltpu.async_copy(sb[t % 2], acc_sh.at[src2[t % 2]], ssem[t % 2],
                         add=True)

    def s_wait(t):
        pltpu.make_async_copy(sb[t % 2], acc_sh.at[src2[t % 2]],
                              ssem[t % 2]).wait()

    _unpack_dst(pk_v, 0, dst2[0], base)
    g_start(0)

    def pair(p, _):
        for t in range(2):
            j = 2 * p + t
            g_wait(t)

            @pl.when(j + 1 < NB)
            def _():
                _unpack_dst(pk_v, j + 1, dst2[(t + 1) % 2], base)
                g_start(t + 1)

            _scale(w_v, j, gb[t % 2], sb[t % 2])
            if t == 0:
                @pl.when(p > 0)
                def _():
                    s_wait(1)
            else:
                s_wait(0)
            _unpack_src(pk_v, j, src2[t % 2])
            s_start(t)
        return 0

    lax.fori_loop(0, NB // 2, pair, 0)
    s_wait(1)


def _zero_acc_slice(gbuf, acc_sh, s):
    _zero_rows(gbuf, B)
    for i in range(RPT // B):
        pltpu.sync_copy(gbuf, acc_sh.at[pl.ds(s * RPT + i * B, B)])
    rem = RPT - (RPT // B) * B
    if rem:
        pltpu.sync_copy(gbuf.at[pl.ds(0, rem)],
                        acc_sh.at[pl.ds(s * RPT + (RPT // B) * B, rem)])


def _sc_degw(pk_hbm, ew_hbm, w_hbm,
             deg_sh, pk_v, w_v, dst0, dtmp_v, ones_v, zb):
    c = lax.axis_index("c")
    s = lax.axis_index("s")

    pltpu.sync_copy(pk_hbm.at[s], pk_v)
    pltpu.sync_copy(ew_hbm.at[s], w_v.at[pl.ds(0, EPT)])

    _zero_rows(zb, B)
    for i in range(RPT // W):
        pltpu.sync_copy(zb.at[i], deg_sh.at[pl.ds(s * RPT + i * W, W)])

    def fill_ones(i, _):
        ones_v[pl.ds(i * 16, 16)] = jnp.full((16,), 1.0, jnp.float32)
        return 0

    lax.fori_loop(0, B // 16, fill_ones, 0)
    plsc.subcore_barrier()

    def deg_body(j, _):
        _unpack_dst(pk_v, j, dst0, 0)
        pltpu.sync_copy(ones_v, deg_sh.at[dst0], add=True)
        return 0

    lax.fori_loop(0, NB, deg_body, 0)
    plsc.subcore_barrier()

    def w_body(j, _):
        _unpack_dst(pk_v, j, dst0, 0)
        pltpu.sync_copy(deg_sh.at[dst0], dtmp_v)
        for k in range(B // 16):
            sl = pl.ds(j * B + k * 16, 16)
            d16 = jnp.maximum(dtmp_v[pl.ds(k * 16, 16)], 1.0)
            w_v[sl] = w_v[sl] / d16
        return 0

    lax.fori_loop(0, NB, w_body, 0)

    @pl.when(c == 0)
    def _():
        pltpu.sync_copy(w_v.at[pl.ds(0, EPT)], w_hbm.at[s])


def _sc_all(pk_hbm, w_hbm, h_hbm, f1_hbm, f2_hbm,
            acc_sh, pk_v, w_v, dst0, dst1,
            src0, src1,
            gb0, gb1, sb0, sb1, gsem0, gsem1, ssem0, ssem1):
    c = lax.axis_index("c")
    s = lax.axis_index("s")
    dst2 = (dst0, dst1)
    src2 = (src0, src1)
    gb = (gb0, gb1)
    sb = (sb0, sb1)
    gsem = (gsem0, gsem1)
    ssem = (ssem0, ssem1)

    pltpu.sync_copy(pk_hbm.at[s], pk_v)
    pltpu.sync_copy(w_hbm.at[s], w_v.at[pl.ds(0, EPT)])

    for r in range(2):
        base = (c + 2 * r) * NP
        _zero_acc_slice(gb0, acc_sh, s)
        plsc.subcore_barrier()
        _spmm_round(pk_v, w_v, dst2, src2, gb, sb, gsem, ssem,
                    h_hbm, acc_sh, base)
        plsc.subcore_barrier()
        pltpu.sync_copy(acc_sh.at[pl.ds(s * RPT, RPT)],
                        f1_hbm.at[pl.ds(base + s * RPT, RPT)])
    plsc.subcore_barrier()

    _zero_acc_slice(gb0, acc_sh, s)
    plsc.subcore_barrier()
    _spmm_round(pk_v, w_v, dst2, src2, gb, sb, gsem, ssem,
                f1_hbm, acc_sh, (2 + c) * NP)
    plsc.subcore_barrier()
    pltpu.sync_copy(acc_sh.at[pl.ds(s * RPT, RPT)],
                    f2_hbm.at[pl.ds(c * NP + s * RPT, RPT)])


_sc_scratch_common = [
    pltpu.VMEM((EPT,), jnp.int32),
    pltpu.VMEM((EPT + 128,), jnp.float32),
    pltpu.VMEM((B,), jnp.int32),
    pltpu.VMEM((B,), jnp.int32),
    pltpu.VMEM((B,), jnp.int32),
    pltpu.VMEM((B,), jnp.int32),
]

_sc_bufs = [
    pltpu.VMEM((B, W), jnp.float32),
    pltpu.VMEM((B, W), jnp.float32),
    pltpu.VMEM((B, W), jnp.float32),
    pltpu.VMEM((B, W), jnp.float32),
    pltpu.SemaphoreType.DMA,
    pltpu.SemaphoreType.DMA,
    pltpu.SemaphoreType.DMA,
    pltpu.SemaphoreType.DMA,
]

_sc_degw_call = pl.kernel(
    _sc_degw,
    out_type=jax.ShapeDtypeStruct((NT, EPT), jnp.float32),
    mesh=_mesh,
    scratch_types=[
        pltpu.VMEM_SHARED((NP,), jnp.float32),
        pltpu.VMEM((EPT,), jnp.int32),
        pltpu.VMEM((EPT + 128,), jnp.float32),
        pltpu.VMEM((B,), jnp.int32),
        pltpu.VMEM((B,), jnp.float32),
        pltpu.VMEM((B,), jnp.float32),
        pltpu.VMEM((B, W), jnp.float32),
    ],
)

_sc_all_call = pl.kernel(
    _sc_all,
    out_type=[
        jax.ShapeDtypeStruct((NC1 * NP, W), jnp.float32),
        jax.ShapeDtypeStruct((NC2 * NP, W), jnp.float32),
    ],
    mesh=_mesh,
    scratch_types=(
        [pltpu.VMEM_SHARED((NP, W), jnp.float32)]
        + _sc_scratch_common
        + _sc_bufs
    ),
)


def _mm_body(x_ref, wt_ref, o_ref):
    o_ref[pl.ds(0, N), :] = jax.lax.dot(
        x_ref[...], wt_ref[...],
        precision=jax.lax.Precision.HIGHEST,
        preferred_element_type=jnp.float32)
    o_ref[pl.ds(N, NP - N), :] = jnp.zeros((NP - N, W), jnp.float32)


def _comb_body(m_ref, f1_ref, f2_ref, h_ref, o_ref):
    scale = 1.0 / 3.0
    m0 = m_ref[0] * scale
    m1 = m_ref[1] * scale
    m2 = m_ref[2] * scale
    o_ref[...] = (m0 * f1_ref[pl.ds(0, N), :]
                  + m1 * f2_ref[pl.ds(0, N), :]
                  + m2 * h_ref[pl.ds(0, N), :])


def kernel(x, edge_index, edge_weight, W0, W1, merger):
    src = edge_index[0]
    dst = edge_index[1]
    pad = EPAD - E
    ar = jnp.arange(pad, dtype=jnp.int32)
    padv = N + (ar % (NP - N))
    dst_p = jnp.concatenate([dst, padv])
    src_p = jnp.concatenate([src, padv])
    packed = (dst_p | (src_p << 16)).reshape(NT, EPT)
    ew_p = jnp.concatenate(
        [edge_weight, jnp.zeros((pad,), jnp.float32)]).reshape(NT, EPT)

    wt = jnp.concatenate([W0.T, W1.T], axis=1)

    h_flat = pl.pallas_call(
        _mm_body,
        grid=(NC1,),
        in_specs=[
            pl.BlockSpec((N, D), lambda c: (0, 0)),
            pl.BlockSpec((D, W), lambda c: (0, c)),
        ],
        out_specs=pl.BlockSpec((NP, W), lambda c: (c, 0)),
        out_shape=jax.ShapeDtypeStruct((NC1 * NP, W), jnp.float32),
    )(x, wt)

    w_pad = _sc_degw_call(packed, ew_p)
    f1_flat, f2_flat = _sc_all_call(packed, w_pad, h_flat)

    out = pl.pallas_call(
        _comb_body,
        grid=(NC2,),
        in_specs=[
            pl.BlockSpec(memory_space=pltpu.SMEM),
            pl.BlockSpec((NP, W), lambda k: (k, 0)),
            pl.BlockSpec((NP, W), lambda k: (k, 0)),
            pl.BlockSpec((NP, W), lambda k: (2 + k, 0)),
        ],
        out_specs=pl.BlockSpec((N, W), lambda k: (0, k)),
        out_shape=jax.ShapeDtypeStruct((N, D), jnp.float32),
    )(merger, f1_flat, f2_flat, h_flat)
    return out
